# baseline (speedup 1.0000x reference)
import functools

import jax
import jax.numpy as jnp
from jax import lax
from jax.experimental import pallas as pl
from jax.experimental.pallas import tpu as pltpu
from jax.experimental.pallas import tpu_sc as plsc

_SC_CORES = 2
_SC_SUBCORES = 16
_SC_WORKERS = _SC_CORES * _SC_SUBCORES

_CHUNK = 128
_ESPLIT = 5
_BE = 1280
_BN = 1000


def _sc_mesh():
    return plsc.VectorSubcoreMesh(core_axis_name="c", subcore_axis_name="s")


def _gather_pairs(h, dst_idx, src_idx):
    n, d = h.shape
    e = dst_idx.shape[1]
    grid = e // _CHUNK

    @functools.partial(
        pl.kernel,
        mesh=_sc_mesh(),
        out_type=[
            jax.ShapeDtypeStruct((e, d), h.dtype),
            jax.ShapeDtypeStruct((e, d), h.dtype),
        ],
    )
    def gk(h_hbm, di_hbm, si_hbm, hd_hbm, hs_hbm):
        def body(di_v, si_v, hd_v, hs_v):
            pltpu.sync_copy(h_hbm.at[di_v.at[0]], hd_v)
            pltpu.sync_copy(h_hbm.at[si_v.at[0]], hs_v)

        pltpu.emit_pipeline(
            body,
            grid=(grid,),
            in_specs=[
                pl.BlockSpec((1, _CHUNK), lambda i: (0, i)),
                pl.BlockSpec((1, _CHUNK), lambda i: (0, i)),
            ],
            out_specs=[
                pl.BlockSpec((_CHUNK, d), lambda i: (i, 0)),
                pl.BlockSpec((_CHUNK, d), lambda i: (i, 0)),
            ],
            core_axis_name=("c", "s"),
            dimension_semantics=(pltpu.PARALLEL,),
        )(di_hbm, si_hbm, hd_hbm, hs_hbm)

    return gk(h, dst_idx, src_idx)


def _segment_sum_partials(msg, dst_idx, zeros_pad):
    e, d = msg.shape
    np_pad = zeros_pad.shape[0]
    rps = np_pad // _SC_SUBCORES
    grid = e // _CHUNK

    @functools.partial(
        pl.kernel,
        mesh=_sc_mesh(),
        out_type=jax.ShapeDtypeStruct((_SC_CORES, np_pad, d), jnp.float32),
        scratch_types=[pltpu.VMEM_SHARED((np_pad, d), jnp.float32)],
    )
    def sk(msg_hbm, di_hbm, z_hbm, out_hbm, acc_sh):
        cid = lax.axis_index("c")
        sid = lax.axis_index("s")
        row0 = sid * rps
        pltpu.sync_copy(z_hbm.at[pl.ds(row0, rps)], acc_sh.at[pl.ds(row0, rps)])
        plsc.subcore_barrier()

        def body(m_v, di_v):
            pltpu.sync_copy(m_v, acc_sh.at[di_v.at[0]], add=True)

        pltpu.emit_pipeline(
            body,
            grid=(grid,),
            in_specs=[
                pl.BlockSpec((_CHUNK, d), lambda i: (i, 0)),
                pl.BlockSpec((1, _CHUNK), lambda i: (0, i)),
            ],
            out_specs=[],
            core_axis_name=("c", "s"),
            dimension_semantics=(pltpu.PARALLEL,),
        )(msg_hbm, di_hbm)

        plsc.subcore_barrier()
        pltpu.sync_copy(acc_sh.at[pl.ds(row0, rps)],
                        out_hbm.at[cid].at[pl.ds(row0, rps)])

    return sk(msg, dst_idx, zeros_pad)


def _mlp_messages(hd, hs, ef16, aux16, w1m, w1a, b1r, w2, b2r):
    e, d = hd.shape
    in3 = 3 * d
    h_dim = w1m.shape[1]
    a_dim = aux16.shape[1]

    def body(hd_ref, hs_ref, ef_ref, aux_ref, w1m_ref, w1a_ref, b1_ref,
             w2_ref, b2_ref, out_ref):
        m = jnp.concatenate(
            [hd_ref[...].astype(jnp.bfloat16),
             hs_ref[...].astype(jnp.bfloat16),
             ef_ref[...]], axis=1)
        hid = jnp.dot(m, w1m_ref[...], preferred_element_type=jnp.float32)
        bias = jnp.dot(aux_ref[...], w1a_ref[...],
                       preferred_element_type=jnp.float32) + b1_ref[...]
        hid = jnp.maximum(hid + bias, 0.0).astype(jnp.bfloat16)
        out_ref[...] = (jnp.dot(hid, w2_ref[...],
                                preferred_element_type=jnp.float32)
                        + b2_ref[...])

    return pl.pallas_call(
        body,
        grid=(e // _BE,),
        in_specs=[
            pl.BlockSpec((_BE, d), lambda i: (i, 0)),
            pl.BlockSpec((_BE, d), lambda i: (i, 0)),
            pl.BlockSpec((_BE, d), lambda i: (i, 0)),
            pl.BlockSpec((1, a_dim), lambda i: (0, 0)),
            pl.BlockSpec((in3, h_dim), lambda i: (0, 0)),
            pl.BlockSpec((a_dim, h_dim), lambda i: (0, 0)),
            pl.BlockSpec((1, h_dim), lambda i: (0, 0)),
            pl.BlockSpec((h_dim, d), lambda i: (0, 0)),
            pl.BlockSpec((1, d), lambda i: (0, 0)),
        ],
        out_specs=pl.BlockSpec((_BE, d), lambda i: (i, 0)),
        out_shape=jax.ShapeDtypeStruct((e, d), jnp.float32),
        compiler_params=pltpu.CompilerParams(
            dimension_semantics=("parallel",)),
    )(hd, hs, ef16, aux16, w1m, w1a, b1r, w2, b2r)


def _gru_update(partials, h, wih_t, whh_t, bihr, bhhr):
    n, d = h.shape
    d3 = wih_t.shape[1]

    np_parts = partials.shape[0]

    def body(p_ref, h_ref, wih_ref, whh_ref, bih_ref, bhh_ref, out_ref):
        a = jnp.sum(p_ref[...], axis=0)
        hv = h_ref[...]
        gi = jnp.dot(a, wih_ref[...],
                     preferred_element_type=jnp.float32) + bih_ref[...]
        gh = jnp.dot(hv, whh_ref[...],
                     preferred_element_type=jnp.float32) + bhh_ref[...]
        i_r, i_z, i_n = gi[:, :d], gi[:, d:2 * d], gi[:, 2 * d:]
        h_r, h_z, h_n = gh[:, :d], gh[:, d:2 * d], gh[:, 2 * d:]
        rg = jax.nn.sigmoid(i_r + h_r)
        z = jax.nn.sigmoid(i_z + h_z)
        nn = jnp.tanh(i_n + rg * h_n)
        out_ref[...] = (1.0 - z) * nn + z * hv

    return pl.pallas_call(
        body,
        grid=(n // _BN,),
        in_specs=[
            pl.BlockSpec((np_parts, _BN, d), lambda i: (0, i, 0)),
            pl.BlockSpec((_BN, d), lambda i: (i, 0)),
            pl.BlockSpec((d, d3), lambda i: (0, 0)),
            pl.BlockSpec((d, d3), lambda i: (0, 0)),
            pl.BlockSpec((1, d3), lambda i: (0, 0)),
            pl.BlockSpec((1, d3), lambda i: (0, 0)),
        ],
        out_specs=pl.BlockSpec((_BN, d), lambda i: (i, 0)),
        out_shape=jax.ShapeDtypeStruct((n, d), jnp.float32),
        compiler_params=pltpu.CompilerParams(
            dimension_semantics=("parallel",)),
    )(partials, h, wih_t, whh_t, bihr, bhhr)


def kernel(node_features, edge_features, edge_index, auxiliary,
           W1, b1, W2, b2, W_ih, b_ih, W_hh, b_hh):
    n, d = node_features.shape
    e = edge_features.shape[0]
    rounds = W1.shape[0]
    in3 = 3 * d

    np_pad = -(-n // (_SC_SUBCORES * 64)) * (_SC_SUBCORES * 64)

    src_idx = edge_index[0].reshape(1, e)
    dst_idx = edge_index[1].reshape(1, e)
    ef16 = edge_features.astype(jnp.bfloat16)
    aux16 = auxiliary.astype(jnp.bfloat16)
    zeros_pad = jnp.zeros((np_pad, d), jnp.float32)

    h = node_features
    for r in range(rounds):
        w1m = W1[r, :in3, :].astype(jnp.bfloat16)
        w1a = W1[r, in3:, :].astype(jnp.bfloat16)
        b1r = b1[r].reshape(1, -1)
        w2r = W2[r].astype(jnp.bfloat16)
        b2r = b2[r].reshape(1, -1)
        wih_t = W_ih[r].T
        whh_t = W_hh[r].T
        bihr = b_ih[r].reshape(1, -1)
        bhhr = b_hh[r].reshape(1, -1)

        ec = e // _ESPLIT
        sls = [slice(k * ec, (k + 1) * ec) for k in range(_ESPLIT)]
        gathered = [_gather_pairs(h, dst_idx[:, sl], src_idx[:, sl])
                    for sl in sls]
        msgs = [_mlp_messages(gathered[k][0], gathered[k][1], ef16[sl],
                              aux16, w1m, w1a, b1r, w2r, b2r)
                for k, sl in enumerate(sls)]
        partials = _segment_sum_partials(jnp.concatenate(msgs, axis=0),
                                         dst_idx, zeros_pad)
        h = _gru_update(partials, h, wih_t, whh_t, bihr, bhhr)
    return h


# speedup vs baseline: 1.1746x; 1.1746x over previous
import functools

import jax
import jax.numpy as jnp
from jax import lax
from jax.experimental import pallas as pl
from jax.experimental.pallas import tpu as pltpu
from jax.experimental.pallas import tpu_sc as plsc

_SC_CORES = 2
_SC_SUBCORES = 16
_SC_WORKERS = _SC_CORES * _SC_SUBCORES

_CHUNK = 128
_ESPLIT = 5
_BE = 1280
_BN = 1000


def _sc_mesh():
    return plsc.VectorSubcoreMesh(core_axis_name="c", subcore_axis_name="s")


def _gather_pairs(h, dst_idx, src_idx):
    n, d = h.shape
    e = dst_idx.shape[1]
    grid = e // _CHUNK

    @functools.partial(
        pl.kernel,
        mesh=_sc_mesh(),
        out_type=[
            jax.ShapeDtypeStruct((e, d), h.dtype),
            jax.ShapeDtypeStruct((e, d), h.dtype),
        ],
    )
    def gk(h_hbm, di_hbm, si_hbm, hd_hbm, hs_hbm):
        def body(di_v, si_v, hd_v, hs_v):
            pltpu.sync_copy(h_hbm.at[di_v.at[0]], hd_v)
            pltpu.sync_copy(h_hbm.at[si_v.at[0]], hs_v)

        pltpu.emit_pipeline(
            body,
            grid=(grid,),
            in_specs=[
                pl.BlockSpec((1, _CHUNK), lambda i: (0, i)),
                pl.BlockSpec((1, _CHUNK), lambda i: (0, i)),
            ],
            out_specs=[
                pl.BlockSpec((_CHUNK, d), lambda i: (i, 0)),
                pl.BlockSpec((_CHUNK, d), lambda i: (i, 0)),
            ],
            core_axis_name=("c", "s"),
            dimension_semantics=(pltpu.PARALLEL,),
        )(di_hbm, si_hbm, hd_hbm, hs_hbm)

    return gk(h, dst_idx, src_idx)


def _segment_sum_partials(msg, dst_idx, zeros_pad):
    e, d = msg.shape
    np_pad = zeros_pad.shape[0]
    rps = np_pad // _SC_SUBCORES
    grid = e // _CHUNK

    @functools.partial(
        pl.kernel,
        mesh=_sc_mesh(),
        out_type=jax.ShapeDtypeStruct((_SC_CORES, np_pad, d), jnp.float32),
        scratch_types=[pltpu.VMEM_SHARED((np_pad, d), jnp.float32)],
    )
    def sk(msg_hbm, di_hbm, z_hbm, out_hbm, acc_sh):
        cid = lax.axis_index("c")
        sid = lax.axis_index("s")
        row0 = sid * rps
        pltpu.sync_copy(z_hbm.at[pl.ds(row0, rps)], acc_sh.at[pl.ds(row0, rps)])
        plsc.subcore_barrier()

        def body(m_v, di_v):
            pltpu.sync_copy(m_v, acc_sh.at[di_v.at[0]], add=True)

        pltpu.emit_pipeline(
            body,
            grid=(grid,),
            in_specs=[
                pl.BlockSpec((_CHUNK, d), lambda i: (i, 0)),
                pl.BlockSpec((1, _CHUNK), lambda i: (0, i)),
            ],
            out_specs=[],
            core_axis_name=("c", "s"),
            dimension_semantics=(pltpu.PARALLEL,),
        )(msg_hbm, di_hbm)

        plsc.subcore_barrier()
        pltpu.sync_copy(acc_sh.at[pl.ds(row0, rps)],
                        out_hbm.at[cid].at[pl.ds(row0, rps)])

    return sk(msg, dst_idx, zeros_pad)


def _mlp_messages(hd, hs, ef16, aux16, w1m, w1a, b1r, w2, b2r):
    e, d = hd.shape
    in3 = 3 * d
    h_dim = w1m.shape[1]
    a_dim = aux16.shape[1]

    def body(hd_ref, hs_ref, ef_ref, aux_ref, w1m_ref, w1a_ref, b1_ref,
             w2_ref, b2_ref, out_ref):
        m = jnp.concatenate(
            [hd_ref[...].astype(jnp.bfloat16),
             hs_ref[...].astype(jnp.bfloat16),
             ef_ref[...]], axis=1)
        hid = jnp.dot(m, w1m_ref[...], preferred_element_type=jnp.float32)
        bias = jnp.dot(aux_ref[...], w1a_ref[...],
                       preferred_element_type=jnp.float32) + b1_ref[...]
        hid = jnp.maximum(hid + bias, 0.0).astype(jnp.bfloat16)
        out_ref[...] = (jnp.dot(hid, w2_ref[...],
                                preferred_element_type=jnp.float32)
                        + b2_ref[...])

    return pl.pallas_call(
        body,
        grid=(e // _BE,),
        in_specs=[
            pl.BlockSpec((_BE, d), lambda i: (i, 0)),
            pl.BlockSpec((_BE, d), lambda i: (i, 0)),
            pl.BlockSpec((_BE, d), lambda i: (i, 0)),
            pl.BlockSpec((1, a_dim), lambda i: (0, 0)),
            pl.BlockSpec((in3, h_dim), lambda i: (0, 0)),
            pl.BlockSpec((a_dim, h_dim), lambda i: (0, 0)),
            pl.BlockSpec((1, h_dim), lambda i: (0, 0)),
            pl.BlockSpec((h_dim, d), lambda i: (0, 0)),
            pl.BlockSpec((1, d), lambda i: (0, 0)),
        ],
        out_specs=pl.BlockSpec((_BE, d), lambda i: (i, 0)),
        out_shape=jax.ShapeDtypeStruct((e, d), jnp.float32),
        compiler_params=pltpu.CompilerParams(
            dimension_semantics=("parallel",)),
    )(hd, hs, ef16, aux16, w1m, w1a, b1r, w2, b2r)


def _gru_update(partials, h, wih_t, whh_t, bihr, bhhr):
    n, d = h.shape
    d3 = wih_t.shape[1]

    np_parts = partials.shape[0]

    def body(p_ref, h_ref, wih_ref, whh_ref, bih_ref, bhh_ref, out_ref):
        a = jnp.sum(p_ref[...], axis=0)
        hv = h_ref[...]
        gi = jnp.dot(a, wih_ref[...],
                     preferred_element_type=jnp.float32) + bih_ref[...]
        gh = jnp.dot(hv, whh_ref[...],
                     preferred_element_type=jnp.float32) + bhh_ref[...]
        i_r, i_z, i_n = gi[:, :d], gi[:, d:2 * d], gi[:, 2 * d:]
        h_r, h_z, h_n = gh[:, :d], gh[:, d:2 * d], gh[:, 2 * d:]
        rg = jax.nn.sigmoid(i_r + h_r)
        z = jax.nn.sigmoid(i_z + h_z)
        nn = jnp.tanh(i_n + rg * h_n)
        out_ref[...] = (1.0 - z) * nn + z * hv

    return pl.pallas_call(
        body,
        grid=(n // _BN,),
        in_specs=[
            pl.BlockSpec((np_parts, _BN, d), lambda i: (0, i, 0)),
            pl.BlockSpec((_BN, d), lambda i: (i, 0)),
            pl.BlockSpec((d, d3), lambda i: (0, 0)),
            pl.BlockSpec((d, d3), lambda i: (0, 0)),
            pl.BlockSpec((1, d3), lambda i: (0, 0)),
            pl.BlockSpec((1, d3), lambda i: (0, 0)),
        ],
        out_specs=pl.BlockSpec((_BN, d), lambda i: (i, 0)),
        out_shape=jax.ShapeDtypeStruct((n, d), jnp.float32),
        compiler_params=pltpu.CompilerParams(
            dimension_semantics=("parallel",)),
    )(partials, h, wih_t, whh_t, bihr, bhhr)


def kernel(node_features, edge_features, edge_index, auxiliary,
           W1, b1, W2, b2, W_ih, b_ih, W_hh, b_hh):
    n, d = node_features.shape
    e = edge_features.shape[0]
    rounds = W1.shape[0]
    in3 = 3 * d

    np_pad = -(-n // (_SC_SUBCORES * 64)) * (_SC_SUBCORES * 64)

    src_idx = edge_index[0].reshape(1, e)
    dst_idx = edge_index[1].reshape(1, e)
    ef16 = edge_features.astype(jnp.bfloat16)
    aux16 = auxiliary.astype(jnp.bfloat16)
    zeros_pad = jnp.zeros((np_pad, d), jnp.float32)

    h = node_features
    for r in range(rounds):
        w1m = W1[r, :in3, :].astype(jnp.bfloat16)
        w1a = W1[r, in3:, :].astype(jnp.bfloat16)
        b1r = b1[r].reshape(1, -1)
        w2r = W2[r].astype(jnp.bfloat16)
        b2r = b2[r].reshape(1, -1)
        wih_t = W_ih[r].T
        whh_t = W_hh[r].T
        bihr = b_ih[r].reshape(1, -1)
        bhhr = b_hh[r].reshape(1, -1)

        ec = e // _ESPLIT
        sls = [slice(k * ec, (k + 1) * ec) for k in range(_ESPLIT)]
        gathered = [_gather_pairs(h, dst_idx[:, sl], src_idx[:, sl])
                    for sl in sls]
        partial_list = []
        for k, sl in enumerate(sls):
            hd, hs = gathered[k]
            msg = _mlp_messages(hd, hs, ef16[sl], aux16,
                                w1m, w1a, b1r, w2r, b2r)
            partial_list.append(
                _segment_sum_partials(msg, dst_idx[:, sl], zeros_pad))
        partials = jnp.concatenate(partial_list, axis=0)
        h = _gru_update(partials, h, wih_t, whh_t, bihr, bhhr)
    return h


# speedup vs baseline: 1.2541x; 1.0677x over previous
import functools

import jax
import jax.numpy as jnp
from jax import lax
from jax.experimental import pallas as pl
from jax.experimental.pallas import tpu as pltpu
from jax.experimental.pallas import tpu_sc as plsc

_SC_CORES = 2
_SC_SUBCORES = 16
_SC_WORKERS = _SC_CORES * _SC_SUBCORES

_CHUNK = 128
_ESPLIT = 5
_BE = 2560
_BN = 1000


def _sc_mesh():
    return plsc.VectorSubcoreMesh(core_axis_name="c", subcore_axis_name="s")


def _gather_pairs(h, dst_idx, src_idx):
    n, d = h.shape
    e = dst_idx.shape[1]
    grid = e // _CHUNK

    @functools.partial(
        pl.kernel,
        mesh=_sc_mesh(),
        out_type=[
            jax.ShapeDtypeStruct((e, d), h.dtype),
            jax.ShapeDtypeStruct((e, d), h.dtype),
        ],
    )
    def gk(h_hbm, di_hbm, si_hbm, hd_hbm, hs_hbm):
        def body(di_v, si_v, hd_v, hs_v):
            pltpu.sync_copy(h_hbm.at[di_v.at[0]], hd_v)
            pltpu.sync_copy(h_hbm.at[si_v.at[0]], hs_v)

        pltpu.emit_pipeline(
            body,
            grid=(grid,),
            in_specs=[
                pl.BlockSpec((1, _CHUNK), lambda i: (0, i)),
                pl.BlockSpec((1, _CHUNK), lambda i: (0, i)),
            ],
            out_specs=[
                pl.BlockSpec((_CHUNK, d), lambda i: (i, 0)),
                pl.BlockSpec((_CHUNK, d), lambda i: (i, 0)),
            ],
            core_axis_name=("c", "s"),
            dimension_semantics=(pltpu.PARALLEL,),
        )(di_hbm, si_hbm, hd_hbm, hs_hbm)

    return gk(h, dst_idx, src_idx)


def _segment_sum_partials(msg, dst_idx, zeros_pad):
    e, d = msg.shape
    np_pad = zeros_pad.shape[0]
    rps = np_pad // _SC_SUBCORES
    grid = e // _CHUNK

    @functools.partial(
        pl.kernel,
        mesh=_sc_mesh(),
        out_type=jax.ShapeDtypeStruct((_SC_CORES, np_pad, d), jnp.float32),
        scratch_types=[pltpu.VMEM_SHARED((np_pad, d), jnp.float32)],
    )
    def sk(msg_hbm, di_hbm, z_hbm, out_hbm, acc_sh):
        cid = lax.axis_index("c")
        sid = lax.axis_index("s")
        row0 = sid * rps
        pltpu.sync_copy(z_hbm.at[pl.ds(row0, rps)], acc_sh.at[pl.ds(row0, rps)])
        plsc.subcore_barrier()

        def body(m_v, di_v):
            pltpu.sync_copy(m_v, acc_sh.at[di_v.at[0]], add=True)

        pltpu.emit_pipeline(
            body,
            grid=(grid,),
            in_specs=[
                pl.BlockSpec((_CHUNK, d), lambda i: (i, 0)),
                pl.BlockSpec((1, _CHUNK), lambda i: (0, i)),
            ],
            out_specs=[],
            core_axis_name=("c", "s"),
            dimension_semantics=(pltpu.PARALLEL,),
        )(msg_hbm, di_hbm)

        plsc.subcore_barrier()
        pltpu.sync_copy(acc_sh.at[pl.ds(row0, rps)],
                        out_hbm.at[cid].at[pl.ds(row0, rps)])

    return sk(msg, dst_idx, zeros_pad)


def _mlp_messages(hd, hs, ef16, aux16, w1m, w1a, b1r, w2, b2r):
    e, d = hd.shape
    in3 = 3 * d
    h_dim = w1m.shape[1]
    a_dim = aux16.shape[1]

    def body(hd_ref, hs_ref, ef_ref, aux_ref, w1m_ref, w1a_ref, b1_ref,
             w2_ref, b2_ref, out_ref):
        m = jnp.concatenate(
            [hd_ref[...].astype(jnp.bfloat16),
             hs_ref[...].astype(jnp.bfloat16),
             ef_ref[...]], axis=1)
        hid = jnp.dot(m, w1m_ref[...], preferred_element_type=jnp.float32)
        bias = jnp.dot(aux_ref[...], w1a_ref[...],
                       preferred_element_type=jnp.float32) + b1_ref[...]
        hid = jnp.maximum(hid + bias, 0.0).astype(jnp.bfloat16)
        out_ref[...] = (jnp.dot(hid, w2_ref[...],
                                preferred_element_type=jnp.float32)
                        + b2_ref[...])

    return pl.pallas_call(
        body,
        grid=(e // _BE,),
        in_specs=[
            pl.BlockSpec((_BE, d), lambda i: (i, 0)),
            pl.BlockSpec((_BE, d), lambda i: (i, 0)),
            pl.BlockSpec((_BE, d), lambda i: (i, 0)),
            pl.BlockSpec((1, a_dim), lambda i: (0, 0)),
            pl.BlockSpec((in3, h_dim), lambda i: (0, 0)),
            pl.BlockSpec((a_dim, h_dim), lambda i: (0, 0)),
            pl.BlockSpec((1, h_dim), lambda i: (0, 0)),
            pl.BlockSpec((h_dim, d), lambda i: (0, 0)),
            pl.BlockSpec((1, d), lambda i: (0, 0)),
        ],
        out_specs=pl.BlockSpec((_BE, d), lambda i: (i, 0)),
        out_shape=jax.ShapeDtypeStruct((e, d), jnp.float32),
        compiler_params=pltpu.CompilerParams(
            dimension_semantics=("parallel",)),
    )(hd, hs, ef16, aux16, w1m, w1a, b1r, w2, b2r)


def _gru_update(partials, h, wih_t, whh_t, bihr, bhhr):
    n, d = h.shape
    d3 = wih_t.shape[1]

    np_parts = partials.shape[0]

    def body(p_ref, h_ref, wih_ref, whh_ref, bih_ref, bhh_ref, out_ref):
        a = jnp.sum(p_ref[...], axis=0)
        hv = h_ref[...]
        gi = jnp.dot(a, wih_ref[...],
                     preferred_element_type=jnp.float32) + bih_ref[...]
        gh = jnp.dot(hv, whh_ref[...],
                     preferred_element_type=jnp.float32) + bhh_ref[...]
        i_r, i_z, i_n = gi[:, :d], gi[:, d:2 * d], gi[:, 2 * d:]
        h_r, h_z, h_n = gh[:, :d], gh[:, d:2 * d], gh[:, 2 * d:]
        rg = jax.nn.sigmoid(i_r + h_r)
        z = jax.nn.sigmoid(i_z + h_z)
        nn = jnp.tanh(i_n + rg * h_n)
        out_ref[...] = (1.0 - z) * nn + z * hv

    return pl.pallas_call(
        body,
        grid=(n // _BN,),
        in_specs=[
            pl.BlockSpec((np_parts, _BN, d), lambda i: (0, i, 0)),
            pl.BlockSpec((_BN, d), lambda i: (i, 0)),
            pl.BlockSpec((d, d3), lambda i: (0, 0)),
            pl.BlockSpec((d, d3), lambda i: (0, 0)),
            pl.BlockSpec((1, d3), lambda i: (0, 0)),
            pl.BlockSpec((1, d3), lambda i: (0, 0)),
        ],
        out_specs=pl.BlockSpec((_BN, d), lambda i: (i, 0)),
        out_shape=jax.ShapeDtypeStruct((n, d), jnp.float32),
        compiler_params=pltpu.CompilerParams(
            dimension_semantics=("parallel",)),
    )(partials, h, wih_t, whh_t, bihr, bhhr)


def kernel(node_features, edge_features, edge_index, auxiliary,
           W1, b1, W2, b2, W_ih, b_ih, W_hh, b_hh):
    n, d = node_features.shape
    e = edge_features.shape[0]
    rounds = W1.shape[0]
    in3 = 3 * d

    np_pad = -(-n // (_SC_SUBCORES * 64)) * (_SC_SUBCORES * 64)

    src_idx = edge_index[0].reshape(1, e)
    dst_idx = edge_index[1].reshape(1, e)
    ef16 = edge_features.astype(jnp.bfloat16)
    aux16 = auxiliary.astype(jnp.bfloat16)
    zeros_pad = jnp.zeros((np_pad, d), jnp.float32)

    h = node_features
    for r in range(rounds):
        w1m = W1[r, :in3, :].astype(jnp.bfloat16)
        w1a = W1[r, in3:, :].astype(jnp.bfloat16)
        b1r = b1[r].reshape(1, -1)
        w2r = W2[r].astype(jnp.bfloat16)
        b2r = b2[r].reshape(1, -1)
        wih_t = W_ih[r].T
        whh_t = W_hh[r].T
        bihr = b_ih[r].reshape(1, -1)
        bhhr = b_hh[r].reshape(1, -1)

        ec = e // _ESPLIT
        sls = [slice(k * ec, (k + 1) * ec) for k in range(_ESPLIT)]
        gathered = [_gather_pairs(h, dst_idx[:, sl], src_idx[:, sl])
                    for sl in sls]
        partial_list = []
        for k, sl in enumerate(sls):
            hd, hs = gathered[k]
            msg = _mlp_messages(hd, hs, ef16[sl], aux16,
                                w1m, w1a, b1r, w2r, b2r)
            partial_list.append(
                _segment_sum_partials(msg, dst_idx[:, sl], zeros_pad))
        partials = jnp.concatenate(partial_list, axis=0)
        h = _gru_update(partials, h, wih_t, whh_t, bihr, bhhr)
    return h


# speedup vs baseline: 1.2730x; 1.0150x over previous
import functools

import jax
import jax.numpy as jnp
from jax import lax
from jax.experimental import pallas as pl
from jax.experimental.pallas import tpu as pltpu
from jax.experimental.pallas import tpu_sc as plsc

_SC_CORES = 2
_SC_SUBCORES = 16
_SC_WORKERS = _SC_CORES * _SC_SUBCORES

_CHUNK = 128
_ESPLIT = 5
_BE = 3200
_BN = 1000


def _sc_mesh():
    return plsc.VectorSubcoreMesh(core_axis_name="c", subcore_axis_name="s")


def _gather_pairs(h, dst_idx, src_idx):
    n, d = h.shape
    e = dst_idx.shape[1]
    grid = e // _CHUNK

    @functools.partial(
        pl.kernel,
        mesh=_sc_mesh(),
        out_type=[
            jax.ShapeDtypeStruct((e, d), h.dtype),
            jax.ShapeDtypeStruct((e, d), h.dtype),
        ],
    )
    def gk(h_hbm, di_hbm, si_hbm, hd_hbm, hs_hbm):
        def body(di_v, si_v, hd_v, hs_v):
            pltpu.sync_copy(h_hbm.at[di_v.at[0]], hd_v)
            pltpu.sync_copy(h_hbm.at[si_v.at[0]], hs_v)

        pltpu.emit_pipeline(
            body,
            grid=(grid,),
            in_specs=[
                pl.BlockSpec((1, _CHUNK), lambda i: (0, i)),
                pl.BlockSpec((1, _CHUNK), lambda i: (0, i)),
            ],
            out_specs=[
                pl.BlockSpec((_CHUNK, d), lambda i: (i, 0)),
                pl.BlockSpec((_CHUNK, d), lambda i: (i, 0)),
            ],
            core_axis_name=("c", "s"),
            dimension_semantics=(pltpu.PARALLEL,),
        )(di_hbm, si_hbm, hd_hbm, hs_hbm)

    return gk(h, dst_idx, src_idx)


def _segment_sum_partials(msg, dst_idx, zeros_pad):
    e, d = msg.shape
    np_pad = zeros_pad.shape[0]
    rps = np_pad // _SC_SUBCORES
    grid = e // _CHUNK

    @functools.partial(
        pl.kernel,
        mesh=_sc_mesh(),
        out_type=jax.ShapeDtypeStruct((_SC_CORES, np_pad, d), jnp.float32),
        scratch_types=[pltpu.VMEM_SHARED((np_pad, d), jnp.float32)],
    )
    def sk(msg_hbm, di_hbm, z_hbm, out_hbm, acc_sh):
        cid = lax.axis_index("c")
        sid = lax.axis_index("s")
        row0 = sid * rps
        pltpu.sync_copy(z_hbm.at[pl.ds(row0, rps)], acc_sh.at[pl.ds(row0, rps)])
        plsc.subcore_barrier()

        def body(m_v, di_v):
            pltpu.sync_copy(m_v, acc_sh.at[di_v.at[0]], add=True)

        pltpu.emit_pipeline(
            body,
            grid=(grid,),
            in_specs=[
                pl.BlockSpec((_CHUNK, d), lambda i: (i, 0)),
                pl.BlockSpec((1, _CHUNK), lambda i: (0, i)),
            ],
            out_specs=[],
            core_axis_name=("c", "s"),
            dimension_semantics=(pltpu.PARALLEL,),
        )(msg_hbm, di_hbm)

        plsc.subcore_barrier()
        pltpu.sync_copy(acc_sh.at[pl.ds(row0, rps)],
                        out_hbm.at[cid].at[pl.ds(row0, rps)])

    return sk(msg, dst_idx, zeros_pad)


def _mlp_messages(hd, hs, ef16, aux16, w1m, w1a, b1r, w2, b2r):
    e, d = hd.shape
    in3 = 3 * d
    h_dim = w1m.shape[1]
    a_dim = aux16.shape[1]

    def body(hd_ref, hs_ref, ef_ref, aux_ref, w1m_ref, w1a_ref, b1_ref,
             w2_ref, b2_ref, out_ref):
        m = jnp.concatenate(
            [hd_ref[...].astype(jnp.bfloat16),
             hs_ref[...].astype(jnp.bfloat16),
             ef_ref[...]], axis=1)
        hid = jnp.dot(m, w1m_ref[...], preferred_element_type=jnp.float32)
        bias = jnp.dot(aux_ref[...], w1a_ref[...],
                       preferred_element_type=jnp.float32) + b1_ref[...]
        hid = jnp.maximum(hid + bias, 0.0).astype(jnp.bfloat16)
        out_ref[...] = (jnp.dot(hid, w2_ref[...],
                                preferred_element_type=jnp.float32)
                        + b2_ref[...])

    return pl.pallas_call(
        body,
        grid=(e // _BE,),
        in_specs=[
            pl.BlockSpec((_BE, d), lambda i: (i, 0)),
            pl.BlockSpec((_BE, d), lambda i: (i, 0)),
            pl.BlockSpec((_BE, d), lambda i: (i, 0)),
            pl.BlockSpec((1, a_dim), lambda i: (0, 0)),
            pl.BlockSpec((in3, h_dim), lambda i: (0, 0)),
            pl.BlockSpec((a_dim, h_dim), lambda i: (0, 0)),
            pl.BlockSpec((1, h_dim), lambda i: (0, 0)),
            pl.BlockSpec((h_dim, d), lambda i: (0, 0)),
            pl.BlockSpec((1, d), lambda i: (0, 0)),
        ],
        out_specs=pl.BlockSpec((_BE, d), lambda i: (i, 0)),
        out_shape=jax.ShapeDtypeStruct((e, d), jnp.float32),
        compiler_params=pltpu.CompilerParams(
            dimension_semantics=("parallel",)),
    )(hd, hs, ef16, aux16, w1m, w1a, b1r, w2, b2r)


def _gru_update(partials, h, wih_t, whh_t, bihr, bhhr):
    n, d = h.shape
    d3 = wih_t.shape[1]

    np_parts = partials.shape[0]

    def body(p_ref, h_ref, wih_ref, whh_ref, bih_ref, bhh_ref, out_ref):
        a = jnp.sum(p_ref[...], axis=0)
        hv = h_ref[...]
        gi = jnp.dot(a, wih_ref[...],
                     preferred_element_type=jnp.float32) + bih_ref[...]
        gh = jnp.dot(hv, whh_ref[...],
                     preferred_element_type=jnp.float32) + bhh_ref[...]
        i_r, i_z, i_n = gi[:, :d], gi[:, d:2 * d], gi[:, 2 * d:]
        h_r, h_z, h_n = gh[:, :d], gh[:, d:2 * d], gh[:, 2 * d:]
        rg = jax.nn.sigmoid(i_r + h_r)
        z = jax.nn.sigmoid(i_z + h_z)
        nn = jnp.tanh(i_n + rg * h_n)
        out_ref[...] = (1.0 - z) * nn + z * hv

    return pl.pallas_call(
        body,
        grid=(n // _BN,),
        in_specs=[
            pl.BlockSpec((np_parts, _BN, d), lambda i: (0, i, 0)),
            pl.BlockSpec((_BN, d), lambda i: (i, 0)),
            pl.BlockSpec((d, d3), lambda i: (0, 0)),
            pl.BlockSpec((d, d3), lambda i: (0, 0)),
            pl.BlockSpec((1, d3), lambda i: (0, 0)),
            pl.BlockSpec((1, d3), lambda i: (0, 0)),
        ],
        out_specs=pl.BlockSpec((_BN, d), lambda i: (i, 0)),
        out_shape=jax.ShapeDtypeStruct((n, d), jnp.float32),
        compiler_params=pltpu.CompilerParams(
            dimension_semantics=("parallel",)),
    )(partials, h, wih_t, whh_t, bihr, bhhr)


def kernel(node_features, edge_features, edge_index, auxiliary,
           W1, b1, W2, b2, W_ih, b_ih, W_hh, b_hh):
    n, d = node_features.shape
    e = edge_features.shape[0]
    rounds = W1.shape[0]
    in3 = 3 * d

    np_pad = -(-n // (_SC_SUBCORES * 64)) * (_SC_SUBCORES * 64)

    src_idx = edge_index[0].reshape(1, e)
    dst_idx = edge_index[1].reshape(1, e)
    ef16 = edge_features.astype(jnp.bfloat16)
    aux16 = auxiliary.astype(jnp.bfloat16)
    zeros_pad = jnp.zeros((np_pad, d), jnp.float32)

    h = node_features
    for r in range(rounds):
        w1m = W1[r, :in3, :].astype(jnp.bfloat16)
        w1a = W1[r, in3:, :].astype(jnp.bfloat16)
        b1r = b1[r].reshape(1, -1)
        w2r = W2[r].astype(jnp.bfloat16)
        b2r = b2[r].reshape(1, -1)
        wih_t = W_ih[r].T
        whh_t = W_hh[r].T
        bihr = b_ih[r].reshape(1, -1)
        bhhr = b_hh[r].reshape(1, -1)

        ec = e // _ESPLIT
        sls = [slice(k * ec, (k + 1) * ec) for k in range(_ESPLIT)]
        gathered = [_gather_pairs(h, dst_idx[:, sl], src_idx[:, sl])
                    for sl in sls]
        partial_list = []
        for k, sl in enumerate(sls):
            hd, hs = gathered[k]
            msg = _mlp_messages(hd, hs, ef16[sl], aux16,
                                w1m, w1a, b1r, w2r, b2r)
            partial_list.append(
                _segment_sum_partials(msg, dst_idx[:, sl], zeros_pad))
        partials = jnp.concatenate(partial_list, axis=0)
        h = _gru_update(partials, h, wih_t, whh_t, bihr, bhhr)
    return h
